# baseline (speedup 1.0000x reference)
import jax
import jax.numpy as jnp
from jax.experimental import pallas as pl
from jax.experimental.pallas import tpu as pltpu


def _pass1_kernel(adj_ref, x_ref, w1_ref, b1_ref, w2_ref,
                  adjb_ref, h2_ref, s1_ref):
    i = pl.program_id(0)

    @pl.when(i == 0)
    def _():
        s1_ref[...] = jnp.dot(x_ref[...], w1_ref[...],
                              preferred_element_type=jnp.float32
                              ).astype(jnp.bfloat16)

    a16 = adj_ref[...].astype(jnp.bfloat16)
    adjb_ref[...] = a16
    h = jnp.dot(a16, s1_ref[...], preferred_element_type=jnp.float32)
    h = jnp.maximum(h + b1_ref[...], 0.0)
    h2_ref[...] = jnp.dot(
        h.astype(jnp.bfloat16), w2_ref[...].astype(jnp.bfloat16),
        preferred_element_type=jnp.float32).astype(jnp.bfloat16)


def _pass2_kernel(adjb_ref, h2_ref, b2_ref, o_ref):
    o = jnp.dot(adjb_ref[...], h2_ref[...],
                preferred_element_type=jnp.float32) + b2_ref[...]
    m = jnp.max(o, axis=1, keepdims=True)
    lse = jnp.log(jnp.sum(jnp.exp(o - m), axis=1, keepdims=True)) + m
    o_ref[...] = o - lse


def kernel(x, adj, W1, b1, W2, b2):
    n, nfeat = x.shape
    nhid = W1.shape[1]
    nclass = W2.shape[1]
    b1r = b1.reshape(1, nhid)
    b2r = b2.reshape(1, nclass)

    bi1 = 200
    adjb, h2 = pl.pallas_call(
        _pass1_kernel,
        grid=(n // bi1,),
        in_specs=[
            pl.BlockSpec((bi1, n), lambda i: (i, 0)),
            pl.BlockSpec((n, nfeat), lambda i: (0, 0)),
            pl.BlockSpec((nfeat, nhid), lambda i: (0, 0)),
            pl.BlockSpec((1, nhid), lambda i: (0, 0)),
            pl.BlockSpec((nhid, nclass), lambda i: (0, 0)),
        ],
        out_specs=[
            pl.BlockSpec((bi1, n), lambda i: (i, 0)),
            pl.BlockSpec((bi1, nclass), lambda i: (i, 0)),
        ],
        out_shape=[
            jax.ShapeDtypeStruct((n, n), jnp.bfloat16),
            jax.ShapeDtypeStruct((n, nclass), jnp.bfloat16),
        ],
        scratch_shapes=[pltpu.VMEM((n, nhid), jnp.bfloat16)],
    )(adj, x, W1, b1r, W2)

    bi2 = 1000
    out = pl.pallas_call(
        _pass2_kernel,
        grid=(n // bi2,),
        in_specs=[
            pl.BlockSpec((bi2, n), lambda i: (i, 0)),
            pl.BlockSpec((n, nclass), lambda i: (0, 0)),
            pl.BlockSpec((1, nclass), lambda i: (0, 0)),
        ],
        out_specs=pl.BlockSpec((bi2, nclass), lambda i: (i, 0)),
        out_shape=jax.ShapeDtypeStruct((n, nclass), jnp.float32),
    )(adjb, h2, b2r)

    return out
